# baseline (speedup 1.0000x reference)
import jax
import jax.numpy as jnp
from jax import lax
from jax.experimental import pallas as pl
from jax.experimental.pallas import tpu as pltpu
from jax.experimental.pallas import tpu_sc as plsc

B = 16384
ROW = 512
IVROW = 640
CHUNK = 32
NWORKERS = 32


def _sc_body(y_hbm, w_hbm, c_hbm, t_hbm, ivy_hbm, ivw_hbm, ivc_hbm,
             yidx_hbm, widx_hbm, cidx_hbm,
             oy_hbm, ow_hbm, oc_hbm,
             data_v, iv_v, idx_v, t_v):
    wid = lax.axis_index("s") * 2 + lax.axis_index("c")

    pltpu.sync_copy(t_hbm, t_v)
    tvec = t_v[...]
    ones = jnp.full((16,), 1, jnp.int32)
    zeros = jnp.full((16,), 0, jnp.int32)
    k = zeros
    for i in range(1, 10):
        k = k + jnp.where(tvec >= 10.0 * i, ones, zeros)

    pltpu.sync_copy(yidx_hbm, idx_v.at[pl.ds(0, 64)])
    pltpu.sync_copy(widx_hbm, idx_v.at[pl.ds(64, 32)])
    pltpu.sync_copy(cidx_hbm, idx_v.at[pl.ds(96, 16)])

    lanes = lax.iota(jnp.int32, 16)
    gcols = [lanes * 10 + k + 160 * ch for ch in range(4)]

    scols_y = [idx_v[pl.ds(16 * ch, 16)] for ch in range(4)]
    w0 = idx_v[pl.ds(64, 16)]
    w1 = idx_v[pl.ds(80, 16)]
    scols_w = [w0, w1, w0 + 256, w1 + 256]
    cidx = idx_v[pl.ds(96, 16)]
    scols_c = [cidx, cidx + 128, cidx + 256, cidx + 384]

    def do_array(in_hbm, iv_hbm, out_hbm, scols, rows_per_worker):
        nchunks = rows_per_worker // CHUNK
        base0 = wid * rows_per_worker

        def chunk_body(i, carry):
            base = base0 + i * CHUNK
            pltpu.sync_copy(in_hbm.at[pl.ds(base, CHUNK)], data_v)
            pltpu.sync_copy(iv_hbm.at[pl.ds(base, CHUNK)], iv_v)

            def row_body(r, c2):
                rv = jnp.full((16,), r, dtype=jnp.int32)
                for ch in range(4):
                    v = plsc.load_gather(iv_v, [rv, gcols[ch]])
                    v = jnp.maximum(v, 0.0)
                    plsc.store_scatter(data_v, [rv, scols[ch]], v)
                return c2

            lax.fori_loop(0, CHUNK, row_body, 0)
            pltpu.sync_copy(data_v, out_hbm.at[pl.ds(base, CHUNK)])
            return carry

        lax.fori_loop(0, nchunks, chunk_body, 0)

    do_array(y_hbm, ivy_hbm, oy_hbm, scols_y, B // NWORKERS)
    do_array(w_hbm, ivw_hbm, ow_hbm, scols_w, B // 2 // NWORKERS)
    do_array(c_hbm, ivc_hbm, oc_hbm, scols_c, B // 4 // NWORKERS)


def kernel(y, w, c, t, iv_y, iv_w, iv_c, y_idx, w_idx, c_idx):
    w2 = w.reshape(B // 2, ROW)
    c2 = c.reshape(B // 4, ROW)
    ivy2 = iv_y.reshape(B, IVROW)
    ivw2 = iv_w.reshape(B // 2, IVROW)
    ivc2 = iv_c.reshape(B // 4, IVROW)
    t16 = jnp.broadcast_to(jnp.reshape(t, (1,)), (16,)).astype(jnp.float32)

    mesh = plsc.VectorSubcoreMesh(core_axis_name="c", subcore_axis_name="s")
    f = pl.kernel(
        _sc_body,
        mesh=mesh,
        out_type=[
            jax.ShapeDtypeStruct((B, ROW), jnp.float32),
            jax.ShapeDtypeStruct((B // 2, ROW), jnp.float32),
            jax.ShapeDtypeStruct((B // 4, ROW), jnp.float32),
        ],
        scratch_types=[
            pltpu.VMEM((CHUNK, ROW), jnp.float32),
            pltpu.VMEM((CHUNK, IVROW), jnp.float32),
            pltpu.VMEM((112,), jnp.int32),
            pltpu.VMEM((16,), jnp.float32),
        ],
        compiler_params=pltpu.CompilerParams(needs_layout_passes=False),
    )
    oy, ow, oc = f(y, w2, c2, t16, ivy2, ivw2, ivc2,
                   y_idx.astype(jnp.int32), w_idx.astype(jnp.int32),
                   c_idx.astype(jnp.int32))
    return (oy, ow.reshape(B, ROW // 2), oc.reshape(B, ROW // 4))
